# speedup vs baseline: 52.7948x; 52.7948x over previous
import numpy as np
import jax
import jax.numpy as jnp
from jax.experimental import pallas as pl
from jax.experimental.pallas import tpu as pltpu

_U_DECAY = 0.97
_X_DECAY = 0.97
_LN_EPS = 1e-5
_L1_EPS = 1e-12


def _ln(x):
    m = jnp.mean(x, axis=-1, keepdims=True)
    v = jnp.mean((x - m) ** 2, axis=-1, keepdims=True)
    return (x - m) * jax.lax.rsqrt(v + _LN_EPS)


def _body(emb_ref, dxt_ref, dyt_ref, et_ref, dm_ref, out_ref, *xu):
    nb = len(xu)
    t, n = xu[0].shape
    emb = emb_ref[...]

    for b in range(nb):
        xu[b][...] = jnp.maximum(
            jnp.dot(emb[b], dxt_ref[...], preferred_element_type=jnp.float32),
            0.0,
        )

    def step(i, carry):
        new = []
        for b in range(nb):
            r = _X_DECAY * carry[b] + xu[b][pl.ds(i, 1), :]
            s = jnp.sum(jnp.abs(r), axis=1, keepdims=True)
            r = r / jnp.maximum(s, _L1_EPS)
            xu[b][pl.ds(i, 1), :] = r
            new.append(r)
        return tuple(new)

    jax.lax.fori_loop(
        0, t, step,
        tuple(jnp.zeros((1, n), jnp.float32) for _ in range(nb)),
    )

    dm = dm_ref[...]
    for b in range(nb):
        x = xu[b][...]
        vl = _ln(emb[b])
        g = jax.lax.dot_general(
            x, x, (((1,), (1,)), ((), ())),
            preferred_element_type=jnp.float32,
        )
        a = jnp.dot(dm * g, vl, preferred_element_type=jnp.float32)
        y = jnp.maximum(
            jnp.dot(_ln(a), dyt_ref[...], preferred_element_type=jnp.float32),
            0.0,
        ) * x
        out_ref[b] = _ln(
            jnp.dot(y, et_ref[...], preferred_element_type=jnp.float32)
        )


def kernel(embeddings, E, Dx, Dy):
    b, t, d = embeddings.shape
    n = E.shape[1]
    nb = 4

    i = np.arange(t)
    expo = np.maximum(i[:, None] - 1 - i[None, :], 0)
    dm = np.where(i[None, :] < i[:, None],
                  np.power(np.float64(_U_DECAY), expo), 0.0).astype(np.float32)

    dxt = Dx.T
    dyt = Dy.T
    et = E.T

    return pl.pallas_call(
        _body,
        grid=(b // nb,),
        in_specs=[
            pl.BlockSpec((nb, t, d), lambda c: (c, 0, 0)),
            pl.BlockSpec((d, n), lambda c: (0, 0)),
            pl.BlockSpec((d, n), lambda c: (0, 0)),
            pl.BlockSpec((n, d), lambda c: (0, 0)),
            pl.BlockSpec((t, t), lambda c: (0, 0)),
        ],
        out_specs=pl.BlockSpec((nb, t, d), lambda c: (c, 0, 0)),
        out_shape=jax.ShapeDtypeStruct((b, t, d), jnp.float32),
        scratch_shapes=[pltpu.VMEM((t, n), jnp.float32) for _ in range(nb)],
        compiler_params=pltpu.CompilerParams(
            dimension_semantics=("parallel",),
            vmem_limit_bytes=48 * 1024 * 1024,
        ),
        name="bdh_fused",
    )(embeddings, jnp.asarray(dxt), jnp.asarray(dyt), jnp.asarray(et),
      jnp.asarray(dm))


# speedup vs baseline: 163.2591x; 3.0923x over previous
import numpy as np
import jax
import jax.numpy as jnp
from jax.experimental import pallas as pl
from jax.experimental.pallas import tpu as pltpu

_U_DECAY = 0.97
_X_DECAY = 0.97
_LN_EPS = 1e-5
_L1_EPS = 1e-12
_NEG = -1e30


def _ln(x):
    m = jnp.mean(x, axis=-1, keepdims=True)
    v = jnp.mean((x - m) ** 2, axis=-1, keepdims=True)
    return (x - m) * jax.lax.rsqrt(v + _LN_EPS)


def _body(emb_ref, dxt_ref, dyt_ref, et_ref, dm_ref, lones_ref, t0s_ref,
          out_ref, xu0, xu1, xu2, xu3, sc_ref, cc_ref):
    xu = (xu0, xu1, xu2, xu3)
    nb = len(xu)
    t, n = xu0.shape
    emb = emb_ref[...]

    for b in range(nb):
        xu[b][...] = jnp.maximum(
            jnp.dot(emb[b], dxt_ref[...], preferred_element_type=jnp.float32),
            0.0,
        )

    sc_ref[:, 0:nb] = jnp.concatenate(
        [jnp.sum(xu[b][...], axis=1, keepdims=True) for b in range(nb)],
        axis=1,
    )

    def sstep(i, sig):
        xi = _X_DECAY * sig + sc_ref[pl.ds(i, 1), :]
        cc_ref[pl.ds(i, 1), :] = jnp.maximum(xi, _L1_EPS)
        return jnp.minimum(xi * (1.0 / _L1_EPS), 1.0)

    jax.lax.fori_loop(0, t, sstep, jnp.zeros((1, 128), jnp.float32),
                      unroll=8)

    cl = jnp.log(cc_ref[...])
    lc = jnp.dot(lones_ref[...], cl,
                 preferred_element_type=jnp.float32)
    lcs_t = jnp.transpose(lc - cl)

    dm = dm_ref[...]
    for b in range(nb):
        u = xu[b][...]
        w0 = jnp.exp(t0s_ref[...] + lcs_t[b:b + 1, :] - lc[:, b:b + 1])
        rcp = 1.0 / cc_ref[:, b:b + 1]
        x = u * rcp + jnp.dot(w0, u, preferred_element_type=jnp.float32)
        xu[b][...] = x

        vl = _ln(emb[b])
        g = jax.lax.dot_general(
            x, x, (((1,), (1,)), ((), ())),
            preferred_element_type=jnp.float32,
        )
        a = jnp.dot(dm * g, vl, preferred_element_type=jnp.float32)
        y = jnp.maximum(
            jnp.dot(_ln(a), dyt_ref[...], preferred_element_type=jnp.float32),
            0.0,
        ) * x
        out_ref[b] = _ln(
            jnp.dot(y, et_ref[...], preferred_element_type=jnp.float32)
        )


def kernel(embeddings, E, Dx, Dy):
    b, t, d = embeddings.shape
    n = E.shape[1]
    nb = 4

    i = np.arange(t)
    expo = np.maximum(i[:, None] - 1 - i[None, :], 0)
    dm = np.where(i[None, :] < i[:, None],
                  np.power(np.float64(_U_DECAY), expo), 0.0).astype(np.float32)
    lones = (i[None, :] <= i[:, None]).astype(np.float32)
    t0s = np.where(i[None, :] < i[:, None],
                   np.log(np.float64(_X_DECAY)) * (i[:, None] - i[None, :]),
                   _NEG).astype(np.float32)

    dxt = Dx.T
    dyt = Dy.T
    et = E.T

    return pl.pallas_call(
        _body,
        grid=(b // nb,),
        in_specs=[
            pl.BlockSpec((nb, t, d), lambda c: (c, 0, 0)),
            pl.BlockSpec((d, n), lambda c: (0, 0)),
            pl.BlockSpec((d, n), lambda c: (0, 0)),
            pl.BlockSpec((n, d), lambda c: (0, 0)),
            pl.BlockSpec((t, t), lambda c: (0, 0)),
            pl.BlockSpec((t, t), lambda c: (0, 0)),
            pl.BlockSpec((t, t), lambda c: (0, 0)),
        ],
        out_specs=pl.BlockSpec((nb, t, d), lambda c: (c, 0, 0)),
        out_shape=jax.ShapeDtypeStruct((b, t, d), jnp.float32),
        scratch_shapes=[pltpu.VMEM((t, n), jnp.float32) for _ in range(nb)]
        + [pltpu.VMEM((t, 128), jnp.float32) for _ in range(2)],
        compiler_params=pltpu.CompilerParams(
            dimension_semantics=("parallel",),
            vmem_limit_bytes=48 * 1024 * 1024,
        ),
        name="bdh_fused",
    )(embeddings, jnp.asarray(dxt), jnp.asarray(dyt), jnp.asarray(et),
      jnp.asarray(dm), jnp.asarray(lones), jnp.asarray(t0s))


# speedup vs baseline: 176.4548x; 1.0808x over previous
import numpy as np
import jax
import jax.numpy as jnp
from jax.experimental import pallas as pl
from jax.experimental.pallas import tpu as pltpu

_U_DECAY = 0.97
_X_DECAY = 0.97
_LN_EPS = 1e-5
_L1_EPS = 1e-12
_NEG = -1e30


def _ln(x):
    m = jnp.mean(x, axis=-1, keepdims=True)
    v = jnp.mean((x - m) ** 2, axis=-1, keepdims=True)
    return (x - m) * jax.lax.rsqrt(v + _LN_EPS)


def _body(emb_ref, dxt_ref, dyt_ref, et_ref, dm_ref, lones_ref, t0s_ref,
          out_ref, xu0, xu1, xu2, xu3, xu4, xu5, xu6, xu7, sc_ref, cc_ref):
    xu = (xu0, xu1, xu2, xu3, xu4, xu5, xu6, xu7)
    nb = len(xu)
    t, n = xu0.shape
    emb = emb_ref[...]

    for b in range(nb):
        xu[b][...] = jnp.maximum(
            jnp.dot(emb[b], dxt_ref[...], preferred_element_type=jnp.float32),
            0.0,
        )

    sc_ref[:, 0:nb] = jnp.concatenate(
        [jnp.sum(xu[b][...], axis=1, keepdims=True) for b in range(nb)],
        axis=1,
    )

    def sstep(i, sig):
        xi = _X_DECAY * sig + sc_ref[pl.ds(i, 1), :]
        cc_ref[pl.ds(i, 1), :] = jnp.maximum(xi, _L1_EPS)
        return jnp.minimum(xi * (1.0 / _L1_EPS), 1.0)

    jax.lax.fori_loop(0, t, sstep, jnp.zeros((1, 128), jnp.float32),
                      unroll=8)

    cl = jnp.log(cc_ref[...])
    lc = jnp.dot(lones_ref[...], cl,
                 preferred_element_type=jnp.float32)
    lcs_t = jnp.transpose(lc - cl)

    dm = dm_ref[...]
    for b in range(nb):
        u = xu[b][...]
        w0 = jnp.exp(t0s_ref[...] + lcs_t[b:b + 1, :] - lc[:, b:b + 1])
        rcp = 1.0 / cc_ref[:, b:b + 1]
        x = u * rcp + jnp.dot(w0, u, preferred_element_type=jnp.float32)
        xu[b][...] = x

        vl = _ln(emb[b])
        g = jax.lax.dot_general(
            x, x, (((1,), (1,)), ((), ())),
            preferred_element_type=jnp.float32,
        )
        a = jnp.dot(dm * g, vl, preferred_element_type=jnp.float32)
        y = jnp.maximum(
            jnp.dot(_ln(a), dyt_ref[...], preferred_element_type=jnp.float32),
            0.0,
        ) * x
        out_ref[b] = _ln(
            jnp.dot(y, et_ref[...], preferred_element_type=jnp.float32)
        )


def kernel(embeddings, E, Dx, Dy):
    b, t, d = embeddings.shape
    n = E.shape[1]
    nb = 8

    i = np.arange(t)
    expo = np.maximum(i[:, None] - 1 - i[None, :], 0)
    dm = np.where(i[None, :] < i[:, None],
                  np.power(np.float64(_U_DECAY), expo), 0.0).astype(np.float32)
    lones = (i[None, :] <= i[:, None]).astype(np.float32)
    t0s = np.where(i[None, :] < i[:, None],
                   np.log(np.float64(_X_DECAY)) * (i[:, None] - i[None, :]),
                   _NEG).astype(np.float32)

    dxt = Dx.T
    dyt = Dy.T
    et = E.T

    return pl.pallas_call(
        _body,
        grid=(b // nb,),
        in_specs=[
            pl.BlockSpec((nb, t, d), lambda c: (c, 0, 0)),
            pl.BlockSpec((d, n), lambda c: (0, 0)),
            pl.BlockSpec((d, n), lambda c: (0, 0)),
            pl.BlockSpec((n, d), lambda c: (0, 0)),
            pl.BlockSpec((t, t), lambda c: (0, 0)),
            pl.BlockSpec((t, t), lambda c: (0, 0)),
            pl.BlockSpec((t, t), lambda c: (0, 0)),
        ],
        out_specs=pl.BlockSpec((nb, t, d), lambda c: (c, 0, 0)),
        out_shape=jax.ShapeDtypeStruct((b, t, d), jnp.float32),
        scratch_shapes=[pltpu.VMEM((t, n), jnp.float32) for _ in range(nb)]
        + [pltpu.VMEM((t, 128), jnp.float32) for _ in range(2)],
        compiler_params=pltpu.CompilerParams(
            dimension_semantics=("arbitrary",),
            vmem_limit_bytes=48 * 1024 * 1024,
        ),
        name="bdh_fused",
    )(embeddings, jnp.asarray(dxt), jnp.asarray(dyt), jnp.asarray(et),
      jnp.asarray(dm), jnp.asarray(lones), jnp.asarray(t0s))
